# Initial kernel scaffold; baseline (speedup 1.0000x reference)
import jax
import jax.numpy as jnp
from jax.experimental import pallas as pl


def kernel(x):
    raise NotImplementedError("write your pallas kernel here")


# speedup vs baseline: 2.7435x; 2.7435x over previous
import functools

import jax
import jax.numpy as jnp
from jax.experimental import pallas as pl
from jax.experimental.pallas import tpu as pltpu

S_BLK = 256
F_BLK = 1024


def _excl_scan(x, s_blk):
    f = x.shape[1]
    zero_row = jnp.zeros((1, f), x.dtype)
    y = jnp.concatenate([zero_row, x[: s_blk - 1]], axis=0)
    k = 1
    while k < s_blk:
        pad = jnp.zeros((k, f), x.dtype)
        y = y + jnp.concatenate([pad, y[: s_blk - k]], axis=0)
        k *= 2
    return y


def _body(x_ref, o_ref, carry_ref, *, s_blk):
    s = pl.program_id(2)

    @pl.when(s == 0)
    def _():
        carry_ref[...] = jnp.zeros_like(carry_ref)

    x = x_ref[0]
    c = carry_ref[...]
    e = _excl_scan(x, s_blk)
    o_ref[0] = e + c
    carry_ref[...] = c + e[s_blk - 1 : s_blk] + x[s_blk - 1 : s_blk]


@jax.jit
def kernel(x):
    b, s, f = x.shape
    ns = s // S_BLK
    nf = f // F_BLK
    return pl.pallas_call(
        functools.partial(_body, s_blk=S_BLK),
        grid=(b, nf, ns),
        in_specs=[
            pl.BlockSpec((1, S_BLK, F_BLK), lambda b, jf, js: (b, js, jf)),
        ],
        out_specs=pl.BlockSpec((1, S_BLK, F_BLK), lambda b, jf, js: (b, js, jf)),
        out_shape=jax.ShapeDtypeStruct((b, s, f), x.dtype),
        scratch_shapes=[pltpu.VMEM((1, F_BLK), x.dtype)],
        compiler_params=pltpu.CompilerParams(
            dimension_semantics=("parallel", "parallel", "arbitrary"),
        ),
    )(x)


# speedup vs baseline: 3.3897x; 1.2356x over previous
import functools

import jax
import jax.numpy as jnp
from jax.experimental import pallas as pl
from jax.experimental.pallas import tpu as pltpu

S_BLK = 256
F_BLK = 2048


def _excl_scan(x, s_blk):
    f = x.shape[1]
    zero_row = jnp.zeros((1, f), x.dtype)
    y = jnp.concatenate([zero_row, x[: s_blk - 1]], axis=0)
    k = 1
    while k < s_blk:
        pad = jnp.zeros((k, f), x.dtype)
        y = y + jnp.concatenate([pad, y[: s_blk - k]], axis=0)
        k *= 2
    return y


def _body(x_ref, o_ref, carry_ref, *, s_blk):
    s = pl.program_id(2)

    @pl.when(s == 0)
    def _():
        carry_ref[...] = jnp.zeros_like(carry_ref)

    x = x_ref[0]
    c = carry_ref[...]
    e = _excl_scan(x, s_blk)
    o_ref[0] = e + c
    carry_ref[...] = c + e[s_blk - 1 : s_blk] + x[s_blk - 1 : s_blk]


@jax.jit
def kernel(x):
    b, s, f = x.shape
    ns = s // S_BLK
    nf = f // F_BLK
    return pl.pallas_call(
        functools.partial(_body, s_blk=S_BLK),
        grid=(b, nf, ns),
        in_specs=[
            pl.BlockSpec((1, S_BLK, F_BLK), lambda b, jf, js: (b, js, jf)),
        ],
        out_specs=pl.BlockSpec((1, S_BLK, F_BLK), lambda b, jf, js: (b, js, jf)),
        out_shape=jax.ShapeDtypeStruct((b, s, f), x.dtype),
        scratch_shapes=[pltpu.VMEM((1, F_BLK), x.dtype)],
        compiler_params=pltpu.CompilerParams(
            dimension_semantics=("parallel", "parallel", "arbitrary"),
        ),
    )(x)


# speedup vs baseline: 4.0994x; 1.2094x over previous
import jax
import jax.numpy as jnp
from jax.experimental import pallas as pl
from jax.experimental.pallas import tpu as pltpu

S_BLK = 256
F_BLK = 2048


def _body(x_ref, l_ref, o_ref, carry_ref):
    s = pl.program_id(2)

    @pl.when(s == 0)
    def _():
        carry_ref[...] = jnp.zeros_like(carry_ref)

    x = x_ref[0]
    hi = x.astype(jnp.bfloat16)
    lo = (x - hi.astype(jnp.float32)).astype(jnp.bfloat16)
    ltri = l_ref[...]
    e = jnp.dot(ltri, hi, preferred_element_type=jnp.float32)
    e = e + jnp.dot(ltri, lo, preferred_element_type=jnp.float32)
    c = carry_ref[...]
    o_ref[0] = e + c
    carry_ref[...] = c + e[S_BLK - 1 : S_BLK] + x[S_BLK - 1 : S_BLK]


@jax.jit
def kernel(x):
    b, s, f = x.shape
    ns = s // S_BLK
    nf = f // F_BLK
    row = jax.lax.broadcasted_iota(jnp.int32, (S_BLK, S_BLK), 0)
    col = jax.lax.broadcasted_iota(jnp.int32, (S_BLK, S_BLK), 1)
    ltri = (col < row).astype(jnp.bfloat16)
    return pl.pallas_call(
        _body,
        grid=(b, nf, ns),
        in_specs=[
            pl.BlockSpec((1, S_BLK, F_BLK), lambda b, jf, js: (b, js, jf)),
            pl.BlockSpec((S_BLK, S_BLK), lambda b, jf, js: (0, 0)),
        ],
        out_specs=pl.BlockSpec((1, S_BLK, F_BLK), lambda b, jf, js: (b, js, jf)),
        out_shape=jax.ShapeDtypeStruct((b, s, f), x.dtype),
        scratch_shapes=[pltpu.VMEM((1, F_BLK), x.dtype)],
        compiler_params=pltpu.CompilerParams(
            dimension_semantics=("parallel", "parallel", "arbitrary"),
        ),
    )(x, ltri)
